# Initial kernel scaffold; baseline (speedup 1.0000x reference)
import jax
import jax.numpy as jnp
from jax.experimental import pallas as pl


def kernel(x):
    raise NotImplementedError("write your pallas kernel here")


# speedup vs baseline: 24.8774x; 24.8774x over previous
import functools

import jax
import jax.numpy as jnp
from jax import lax
from jax.experimental import pallas as pl
from jax.experimental.pallas import tpu as pltpu
from jax.experimental.pallas import tpu_sc as plsc

TIME = 16
MAXLAT = 15
LANES = 16


def _spike_sc(xf, B, N):
    info = plsc.get_sparse_core_info()
    NC, NS = info.num_cores, info.num_subcores
    NW = NC * NS
    halves = NW // B
    span = N // halves
    CH = 3584
    n_chunks = span // CH
    assert span % CH == 0 and CH % LANES == 0

    mesh = plsc.VectorSubcoreMesh(core_axis_name="c", subcore_axis_name="s")

    @functools.partial(
        pl.kernel,
        mesh=mesh,
        out_type=jax.ShapeDtypeStruct((B, TIME, N), jnp.float32),
        scratch_types=[
            pltpu.VMEM((CH,), jnp.float32),
            pltpu.VMEM((TIME, CH), jnp.float32),
        ],
    )
    def body(x_hbm, out_hbm, x_v, o_v):
        wid = lax.axis_index("s") * NC + lax.axis_index("c")
        b = wid // halves
        base = (wid % halves) * span

        def chunk_body(ci, carry):
            n0 = base + ci * CH
            pltpu.sync_copy(x_hbm.at[b, pl.ds(n0, CH)], x_v)

            def group(g, carry2):
                xv = x_v[pl.ds(g * LANES, LANES)]
                xn = jnp.minimum(jnp.maximum(xv, 0.0), 1.0)
                lat = jnp.minimum(
                    jnp.maximum(((1.0 - xn) * float(MAXLAT)).astype(jnp.int32), 0),
                    TIME - 1,
                )
                latx = jnp.where(xn > 0.0, lat, TIME)
                one = jnp.full((LANES,), 1.0, jnp.float32)
                zero = jnp.zeros((LANES,), jnp.float32)
                for t in range(TIME):
                    o_v[t, pl.ds(g * LANES, LANES)] = jnp.where(latx == t, one, zero)
                return carry2

            lax.fori_loop(0, CH // LANES, group, 0)
            pltpu.sync_copy(o_v, out_hbm.at[b, :, pl.ds(n0, CH)])
            return carry

        lax.fori_loop(0, n_chunks, chunk_body, 0)

    return body


def kernel(x):
    B = x.shape[0]
    feat = x.shape[1:]
    N = 1
    for d in feat:
        N *= d
    xf = x.reshape(B, N)
    out = _spike_sc(xf, B, N)(xf)
    return out.reshape((B, TIME) + feat)


# speedup vs baseline: 32.3786x; 1.3015x over previous
import functools

import jax
import jax.numpy as jnp
from jax import lax
from jax.experimental import pallas as pl
from jax.experimental.pallas import tpu as pltpu
from jax.experimental.pallas import tpu_sc as plsc

TIME = 16
MAXLAT = 15
LANES = 16
CH = 2688
UNROLL = 4


def _spike_sc(B, N):
    info = plsc.get_sparse_core_info()
    NC, NS = info.num_cores, info.num_subcores
    NW = NC * NS
    halves = NW // B
    span = N // halves
    n_chunks = span // CH
    groups = CH // LANES
    assert span % CH == 0 and groups % UNROLL == 0

    mesh = plsc.VectorSubcoreMesh(core_axis_name="c", subcore_axis_name="s")

    @functools.partial(
        pl.kernel,
        mesh=mesh,
        out_type=jax.ShapeDtypeStruct((B * TIME * N,), jnp.float32),
        compiler_params=pltpu.CompilerParams(needs_layout_passes=False),
        scratch_types=[
            pltpu.VMEM((CH,), jnp.float32),
            pltpu.VMEM((CH,), jnp.float32),
            pltpu.VMEM(((TIME + 1) * CH,), jnp.float32),
            pltpu.VMEM(((TIME + 1) * CH,), jnp.float32),
            pltpu.VMEM((CH,), jnp.int32),
            pltpu.VMEM((CH,), jnp.int32),
            pltpu.SemaphoreType.DMA,
            pltpu.SemaphoreType.DMA,
            pltpu.SemaphoreType.DMA,
            pltpu.SemaphoreType.DMA,
        ],
    )
    def body(x_hbm, out_hbm, x0, x1, o0, o1, i0, i1, sin0, sin1, sout0, sout1):
        wid = lax.axis_index("s") * NC + lax.axis_index("c")
        b = wid // halves
        base = (wid % halves) * span
        flat0 = b * N + base

        xb = [x0, x1]
        ob = [o0, o1]
        ib = [i0, i1]
        sin = [sin0, sin1]
        sout = [sout0, sout1]

        iota = jnp.arange(LANES, dtype=jnp.int32)
        ones = jnp.full((LANES,), 1.0, jnp.float32)
        zerov = jnp.zeros((LANES,), jnp.float32)
        sent = jnp.full((LANES,), TIME, jnp.int32)

        sinkbase = sent * CH + iota
        for s in range(2):
            o_s, i_s = ob[s], ib[s]

            def zcol(i, c2):
                o_s[pl.ds(i * LANES, LANES)] = zerov
                return c2

            lax.fori_loop(0, (TIME + 1) * groups, zcol, 0)

            def isent(i, c):
                i_s[pl.ds(i * LANES, LANES)] = sinkbase
                return c

            lax.fori_loop(0, groups, isent, 0)

        h_in = [None, None]
        h_out = [None, None]
        for ci in range(n_chunks):
            s = ci % 2
            if ci == 0:
                h_in[0] = pltpu.async_copy(
                    x_hbm.at[pl.ds(flat0, CH)], xb[0], sin[0]
                )
            if ci + 1 < n_chunks:
                sn = (ci + 1) % 2
                h_in[sn] = pltpu.async_copy(
                    x_hbm.at[pl.ds(flat0 + (ci + 1) * CH, CH)], xb[sn], sin[sn]
                )
            h_in[s].wait()
            if ci >= 2:
                for h in h_out[s]:
                    h.wait()

            x_s, o_s, i_s = xb[s], ob[s], ib[s]

            def clear(i, c):
                for u in range(UNROLL):
                    g = i * UNROLL + u
                    old = i_s[pl.ds(g * LANES, LANES)]
                    plsc.store_scatter(o_s, [old], zerov)
                return c

            lax.fori_loop(0, groups // UNROLL, clear, 0)

            def scat(i, c):
                for u in range(UNROLL):
                    g = i * UNROLL + u
                    off = iota + g * LANES
                    xv = x_s[pl.ds(g * LANES, LANES)]
                    xn = jnp.minimum(jnp.maximum(xv, 0.0), 1.0)
                    lat = ((1.0 - xn) * float(MAXLAT)).astype(jnp.int32)
                    lat = jnp.minimum(lat, TIME - 1)
                    latx = jnp.where(xn > 0.0, lat, sent)
                    flat = latx * CH + off
                    plsc.store_scatter(o_s, [flat], ones)
                    i_s[pl.ds(g * LANES, LANES)] = flat
                return c

            lax.fori_loop(0, groups // UNROLL, scat, 0)

            n0 = base + ci * CH
            h_out[s] = [
                pltpu.async_copy(
                    o_s.at[pl.ds(t * CH, CH)],
                    out_hbm.at[pl.ds(b * (TIME * N) + t * N + n0, CH)],
                    sout[s],
                )
                for t in range(TIME)
            ]

        for h in h_out[(n_chunks - 2) % 2]:
            h.wait()
        for h in h_out[(n_chunks - 1) % 2]:
            h.wait()

    return body


def kernel(x):
    B = x.shape[0]
    feat = x.shape[1:]
    N = 1
    for d in feat:
        N *= d
    xf = x.reshape(B * N)
    out = _spike_sc(B, N)(xf)
    return out.reshape((B, TIME) + feat)


# speedup vs baseline: 106.0279x; 3.2746x over previous
import functools

import jax
import jax.numpy as jnp
from jax import lax
from jax.experimental import pallas as pl
from jax.experimental.pallas import tpu as pltpu
from jax.experimental.pallas import tpu_sc as plsc

TIME = 16
MAXLAT = 15
LANES = 16
HB = 8


def _spike_sc(B, C, H, W):
    info = plsc.get_sparse_core_info()
    NC, NS = info.num_cores, info.num_subcores
    NW = NC * NS
    hblocks = H // HB
    blocks_c = hblocks * C
    n_blocks = B * blocks_c
    per_w = n_blocks // NW
    assert n_blocks % NW == 0 and per_w % 2 == 0 and W % LANES == 0
    wgroups = W // LANES

    mesh = plsc.VectorSubcoreMesh(core_axis_name="c", subcore_axis_name="s")

    @functools.partial(
        pl.kernel,
        mesh=mesh,
        out_type=jax.ShapeDtypeStruct((B, TIME, C, H, W), jnp.float32),
        compiler_params=pltpu.CompilerParams(needs_layout_passes=False),
        scratch_types=[
            pltpu.VMEM((HB, W), jnp.float32),
            pltpu.VMEM((HB, W), jnp.float32),
            pltpu.VMEM((TIME, HB, W), jnp.float32),
            pltpu.VMEM((TIME, HB, W), jnp.float32),
            pltpu.SemaphoreType.DMA,
            pltpu.SemaphoreType.DMA,
            pltpu.SemaphoreType.DMA,
            pltpu.SemaphoreType.DMA,
        ],
    )
    def body(x_hbm, out_hbm, x0, x1, o0, o1, sin0, sin1, sout0, sout1):
        wid = lax.axis_index("s") * NC + lax.axis_index("c")
        k0 = wid * per_w

        def decode(k):
            b = k // blocks_c
            r = k - b * blocks_c
            c = r // hblocks
            h0 = (r - c * hblocks) * HB
            return b, c, h0

        def issue_in(k, x_s, sem):
            b, c, h0 = decode(k)
            return pltpu.async_copy(
                x_hbm.at[b, c, pl.ds(h0, HB)], x_s, sem
            )

        def wait_in(x_s, sem):
            pltpu.make_async_copy(x_hbm.at[0, 0, pl.ds(0, HB)], x_s, sem).wait()

        def wait_out(o_s, sem):
            pltpu.make_async_copy(
                o_s, out_hbm.at[0, :, 0, pl.ds(0, HB)], sem
            ).wait()

        def compute(x_s, o_s):
            def wg_body(g, carry):
                w0 = g * LANES
                for r in range(HB):
                    xv = x_s[r, pl.ds(w0, LANES)]
                    xn = jnp.minimum(jnp.maximum(xv, 0.0), 1.0)
                    lat = ((1.0 - xn) * float(MAXLAT)).astype(jnp.int32)
                    lat = jnp.minimum(lat, TIME - 1)
                    latx = jnp.where(xn > 0.0, lat, TIME)
                    for t in range(TIME):
                        o_s[t, r, pl.ds(w0, LANES)] = jnp.where(
                            latx == t, 1.0, 0.0
                        )
                return carry

            lax.fori_loop(0, wgroups, wg_body, 0)

        def issue_out(k, o_s, sem):
            b, c, h0 = decode(k)
            return pltpu.async_copy(
                o_s, out_hbm.at[b, :, c, pl.ds(h0, HB)], sem
            )

        issue_in(k0, x0, sin0)
        issue_in(k0 + 1, x1, sin1)

        def pair(j, carry):
            ka = k0 + 2 * j
            wait_in(x0, sin0)

            @pl.when(j > 0)
            def _():
                wait_out(o0, sout0)

            compute(x0, o0)
            issue_out(ka, o0, sout0)

            @pl.when(2 * j + 2 < per_w)
            def _():
                issue_in(ka + 2, x0, sin0)

            wait_in(x1, sin1)

            @pl.when(j > 0)
            def _():
                wait_out(o1, sout1)

            compute(x1, o1)
            issue_out(ka + 1, o1, sout1)

            @pl.when(2 * j + 3 < per_w)
            def _():
                issue_in(ka + 3, x1, sin1)

            return carry

        lax.fori_loop(0, per_w // 2, pair, 0)
        wait_out(o0, sout0)
        wait_out(o1, sout1)

    return body


def kernel(x):
    B, C, H, W = x.shape
    return _spike_sc(B, C, H, W)(x)
